# Initial kernel scaffold; baseline (speedup 1.0000x reference)
import jax
import jax.numpy as jnp
from jax.experimental import pallas as pl


def kernel(x):
    raise NotImplementedError("write your pallas kernel here")


# speedup vs baseline: 1.0287x; 1.0287x over previous
import jax
import jax.numpy as jnp
from jax.experimental import pallas as pl

_BLOCK_ROWS = 1024


def _log_assign_body(x_ref, o_ref):
    o_ref[...] = jnp.log(x_ref[...])

    @pl.when(pl.program_id(0) == 0)
    def _():
        blk = o_ref[0:8, :]
        rows = jax.lax.broadcasted_iota(jnp.int32, blk.shape, 0)
        cols = jax.lax.broadcasted_iota(jnp.int32, blk.shape, 1)
        blk = jnp.where(rows == 2, jnp.float32(1.0), blk)
        blk = jnp.where((rows == 1) & (cols == 1), jnp.float32(5.0), blk)
        o_ref[0:8, :] = blk


def kernel(x):
    n_rows, n_cols = x.shape
    grid = (n_rows // _BLOCK_ROWS,)
    return pl.pallas_call(
        _log_assign_body,
        grid=grid,
        in_specs=[pl.BlockSpec((_BLOCK_ROWS, n_cols), lambda i: (i, 0))],
        out_specs=pl.BlockSpec((_BLOCK_ROWS, n_cols), lambda i: (i, 0)),
        out_shape=jax.ShapeDtypeStruct((n_rows, n_cols), x.dtype),
    )(x)


# speedup vs baseline: 1.0612x; 1.0316x over previous
import jax
import jax.numpy as jnp
from jax.experimental import pallas as pl

_BLOCK_ROWS = 2048


def _log_assign_body(x_ref, o_ref):
    o_ref[...] = jnp.log(x_ref[...])

    @pl.when(pl.program_id(0) == 0)
    def _():
        blk = o_ref[0:8, :]
        rows = jax.lax.broadcasted_iota(jnp.int32, blk.shape, 0)
        cols = jax.lax.broadcasted_iota(jnp.int32, blk.shape, 1)
        blk = jnp.where(rows == 2, jnp.float32(1.0), blk)
        blk = jnp.where((rows == 1) & (cols == 1), jnp.float32(5.0), blk)
        o_ref[0:8, :] = blk


def kernel(x):
    n_rows, n_cols = x.shape
    grid = (n_rows // _BLOCK_ROWS,)
    return pl.pallas_call(
        _log_assign_body,
        grid=grid,
        in_specs=[pl.BlockSpec((_BLOCK_ROWS, n_cols), lambda i: (i, 0))],
        out_specs=pl.BlockSpec((_BLOCK_ROWS, n_cols), lambda i: (i, 0)),
        out_shape=jax.ShapeDtypeStruct((n_rows, n_cols), x.dtype),
    )(x)


# speedup vs baseline: 1.0801x; 1.0178x over previous
import jax
import jax.numpy as jnp
from jax.experimental import pallas as pl
from jax.experimental.pallas import tpu as pltpu

_BLOCK_ROWS = 1024
_BUFFER_COUNT = 4


def _patch_first_rows(o_blk):
    blk = o_blk[0:8, :]
    rows = jax.lax.broadcasted_iota(jnp.int32, blk.shape, 0)
    cols = jax.lax.broadcasted_iota(jnp.int32, blk.shape, 1)
    blk = jnp.where(rows == 2, jnp.float32(1.0), blk)
    blk = jnp.where((rows == 1) & (cols == 1), jnp.float32(5.0), blk)
    o_blk[0:8, :] = blk


def _outer(x_hbm, o_hbm):
    n_rows, n_cols = x_hbm.shape

    def _inner(idx, x_blk, o_blk):
        (i,) = idx
        o_blk[...] = jnp.log(x_blk[...])

        @pl.when(i == 0)
        def _():
            _patch_first_rows(o_blk)

    in_spec = pl.BlockSpec(
        (_BLOCK_ROWS, n_cols),
        lambda i: (i, 0),
        pipeline_mode=pl.Buffered(buffer_count=_BUFFER_COUNT),
    )
    out_spec = pl.BlockSpec((_BLOCK_ROWS, n_cols), lambda i: (i, 0))
    pipe = pltpu.emit_pipeline(
        _inner,
        grid=(n_rows // _BLOCK_ROWS,),
        in_specs=[in_spec],
        out_specs=[out_spec],
        _explicit_indices=True,
    )
    pipe(x_hbm, o_hbm)


def kernel(x):
    n_rows, n_cols = x.shape
    return pl.pallas_call(
        _outer,
        in_specs=[pl.BlockSpec(memory_space=pl.ANY)],
        out_specs=pl.BlockSpec(memory_space=pl.ANY),
        out_shape=jax.ShapeDtypeStruct((n_rows, n_cols), x.dtype),
    )(x)
